# baseline (speedup 1.0000x reference)
import functools
import math

import jax
import jax.numpy as jnp
from jax import lax
from jax.experimental import pallas as pl
from jax.experimental.pallas import tpu as pltpu
from jax.experimental.pallas import tpu_sc as plsc

D_MODEL = 64
SCALE = math.sqrt(D_MODEL)

NUM_CORES = 2
NUM_SUBCORES = 16
NW = NUM_CORES * NUM_SUBCORES
LANES = 16

SPLIT = 128
NBUF = 4


def _make_kernel(n_rows, row_len, d):
    assert n_rows % (NW * NBUF) == 0 and row_len % 8 == 0
    rows_per_w = n_rows // NW
    n_steps = rows_per_w // NBUF
    splits = ((0, SPLIT), (SPLIT, row_len - SPLIT))
    pairs = row_len // 2

    mesh = plsc.VectorSubcoreMesh(core_axis_name="c", subcore_axis_name="s")

    @functools.partial(
        pl.kernel,
        out_type=jax.ShapeDtypeStruct((n_rows * pairs, 2 * d), jnp.float32),
        mesh=mesh,
        scratch_types=(
            [pltpu.VMEM((rows_per_w * row_len,), jnp.int32)]
            + [pltpu.VMEM((row_len, d), jnp.float32) for _ in range(NBUF)]
            + [pltpu.VMEM((pairs, 2 * d), jnp.float32) for _ in range(NBUF)]
            + [pltpu.SemaphoreType.DMA for _ in range(2 * NBUF)]
        ),
        compiler_params=pltpu.CompilerParams(use_tc_tiling_on_sc=False),
    )
    def emb_kernel(idx_hbm, lut_hbm, out_hbm, idx_v, *bufs_and_sems):
        gbufs = bufs_and_sems[:NBUF]
        sbufs = bufs_and_sems[NBUF:2 * NBUF]
        gsems = bufs_and_sems[2 * NBUF:3 * NBUF]
        ssems = bufs_and_sems[3 * NBUF:]
        wid = lax.axis_index("s") * NUM_CORES + lax.axis_index("c")
        row_base = wid * rows_per_w

        pltpu.sync_copy(
            idx_hbm.at[pl.ds(row_base * row_len, rows_per_w * row_len)], idx_v
        )

        def fire_gather(g, b):
            for lo, ln in splits:
                pltpu.async_copy(
                    lut_hbm.at[idx_v.at[pl.ds(g * row_len + lo, ln)]],
                    gbufs[b].at[pl.ds(lo, ln)],
                    gsems[b],
                )

        def wait_gather(b):
            pltpu.make_async_copy(
                lut_hbm.at[pl.ds(0, row_len)], gbufs[b], gsems[b]
            ).wait()

        def scale_repack(b):
            def body(i, c):
                for rr in range(8):
                    r = i * 8 + rr
                    p = i * 4 + rr // 2
                    off = (rr % 2) * d
                    for cc in range(d // LANES):
                        sbufs[b][p, pl.ds(off + cc * LANES, LANES)] = (
                            gbufs[b][r, pl.ds(cc * LANES, LANES)] * SCALE
                        )
                return c

            lax.fori_loop(0, row_len // 8, body, 0, unroll=False)

        def fire_scatter(g, b):
            pltpu.async_copy(
                sbufs[b],
                out_hbm.at[pl.ds((row_base + g) * pairs, pairs)],
                ssems[b],
            )

        def wait_scatter(b):
            pltpu.make_async_copy(
                sbufs[b], out_hbm.at[pl.ds(0, pairs)], ssems[b]
            ).wait()

        def process(g, b, wait_sct, fire_ahd):
            wait_gather(b)
            if wait_sct:
                wait_scatter(b)
            scale_repack(b)
            fire_scatter(g, b)
            if fire_ahd:
                fire_gather(g + NBUF - 1, (b + NBUF - 1) % NBUF)

        for g in range(NBUF - 1):
            fire_gather(g, g)

        for b in range(NBUF):
            process(b, b, wait_sct=False, fire_ahd=True)

        def step(s, carry):
            for b in range(NBUF):
                process(s * NBUF + b, b, wait_sct=True, fire_ahd=True)
            return carry

        lax.fori_loop(1, n_steps - 1, step, 0, unroll=False)

        g0 = (n_steps - 1) * NBUF
        process(g0, 0, wait_sct=True, fire_ahd=True)
        for b in range(1, NBUF):
            process(g0 + b, b, wait_sct=True, fire_ahd=False)

        for b in range(NBUF):
            wait_scatter(b)

    return emb_kernel


@jax.jit
def kernel(x, lut):
    b0, b1 = x.shape
    d = lut.shape[1]
    out2d = _make_kernel(b0, b1, d)(x.reshape(-1), lut)
    return out2d.reshape(b0, b1, d)


# speedup vs baseline: 1.3294x; 1.3294x over previous
import functools
import math

import jax
import jax.numpy as jnp
from jax import lax
from jax.experimental import pallas as pl
from jax.experimental.pallas import tpu as pltpu
from jax.experimental.pallas import tpu_sc as plsc

D_MODEL = 64
SCALE = math.sqrt(D_MODEL)

NUM_CORES = 2
NUM_SUBCORES = 16
NW = NUM_CORES * NUM_SUBCORES
LANES = 16

SPLIT = 128
NBUF = 4


def _make_kernel(n_rows, row_len, d):
    assert n_rows % (NW * NBUF) == 0 and row_len % 8 == 0
    rows_per_w = n_rows // NW
    n_steps = rows_per_w // NBUF
    splits = ((0, SPLIT), (SPLIT, row_len - SPLIT))

    mesh = plsc.VectorSubcoreMesh(core_axis_name="c", subcore_axis_name="s")

    @functools.partial(
        pl.kernel,
        out_type=jax.ShapeDtypeStruct((n_rows * row_len, 2 * d), jnp.float32),
        mesh=mesh,
        scratch_types=(
            [pltpu.VMEM((rows_per_w * row_len,), jnp.int32)]
            + [pltpu.VMEM((row_len, d), jnp.float32) for _ in range(NBUF)]
            + [pltpu.SemaphoreType.DMA for _ in range(2 * NBUF)]
        ),
        compiler_params=pltpu.CompilerParams(use_tc_tiling_on_sc=False),
    )
    def emb_kernel(idx_hbm, lut_hbm, out_hbm, idx_v, *bufs_and_sems):
        bufs = bufs_and_sems[:NBUF]
        gsems = bufs_and_sems[NBUF:2 * NBUF]
        ssems = bufs_and_sems[2 * NBUF:]
        wid = lax.axis_index("s") * NUM_CORES + lax.axis_index("c")
        row_base = wid * rows_per_w

        pltpu.sync_copy(
            idx_hbm.at[pl.ds(row_base * row_len, rows_per_w * row_len)], idx_v
        )

        def fire_gather(g, b):
            for lo, ln in splits:
                pltpu.async_copy(
                    lut_hbm.at[idx_v.at[pl.ds(g * row_len + lo, ln)]],
                    bufs[b].at[pl.ds(lo, ln)],
                    gsems[b],
                )

        def wait_gather(b):
            pltpu.make_async_copy(
                lut_hbm.at[pl.ds(0, row_len)], bufs[b], gsems[b]
            ).wait()

        def scale(b):
            def body(i, c):
                for rr in range(8):
                    r = i * 8 + rr
                    for cc in range(d // LANES):
                        sl = pl.ds(cc * LANES, LANES)
                        bufs[b][r, sl] = bufs[b][r, sl] * SCALE
                return c

            lax.fori_loop(0, row_len // 8, body, 0, unroll=False)

        def out_dst(g):
            return out_hbm.at[
                pl.ds((row_base + g) * row_len, row_len), pl.ds(0, d)
            ]

        def fire_scatter(g, b):
            pltpu.async_copy(bufs[b], out_dst(g), ssems[b])

        def wait_scatter(b):
            pltpu.make_async_copy(bufs[b], out_dst(0), ssems[b]).wait()

        def process(g, b, wait_sct, fire_ahd):
            wait_gather(b)
            if wait_sct:
                wait_scatter(b)
            scale(b)
            fire_scatter(g, b)
            if fire_ahd:
                fire_gather(g + NBUF - 1, (b + NBUF - 1) % NBUF)

        for g in range(NBUF - 1):
            fire_gather(g, g)

        for b in range(NBUF):
            process(b, b, wait_sct=False, fire_ahd=True)

        def step(s, carry):
            for b in range(NBUF):
                process(s * NBUF + b, b, wait_sct=True, fire_ahd=True)
            return carry

        lax.fori_loop(1, n_steps - 1, step, 0, unroll=False)

        g0 = (n_steps - 1) * NBUF
        process(g0, 0, wait_sct=True, fire_ahd=True)
        for b in range(1, NBUF):
            process(g0 + b, b, wait_sct=True, fire_ahd=False)

        for b in range(NBUF):
            wait_scatter(b)

    return emb_kernel


@jax.jit
def kernel(x, lut):
    b0, b1 = x.shape
    d = lut.shape[1]
    out_pad = _make_kernel(b0, b1, d)(x.reshape(-1), lut)
    return out_pad[:, :d].reshape(b0, b1, d)
